# baseline (speedup 1.0000x reference)
import functools

import jax
import jax.numpy as jnp
from jax import lax
from jax.experimental import pallas as pl
from jax.experimental.pallas import tpu as pltpu
from jax.experimental.pallas import tpu_sc as plsc

N = 16384
Q = 1024
NSUB = 16
CHUNK = N // NSUB
VECS = CHUNK // 16
STRIPE = CHUNK // 16
QPW = Q // NSUB
NBLK = Q // QPW
ABLK = 3 * QPW
CBLK = 4 * QPW
SMOOTH = 0.1
LN2 = 0.6931471805599453


def _vlog(x):
    xi = lax.bitcast_convert_type(x, jnp.int32)
    e = lax.shift_right_arithmetic(xi, 23) - 127
    m = lax.bitcast_convert_type(
        (xi & 0x007FFFFF) | 0x3F800000, jnp.float32)
    big = m >= 1.4142135623730951
    m = jnp.where(big, m * 0.5, m)
    e = jnp.where(big, e + 1, e)
    f = m - 1.0
    s = f / (2.0 + f)
    w = s * s
    p = w * (0.6666666666666735 + w * (0.3999999999940942
         + w * (0.2857142874366239 + w * 0.22222198432149784)))
    return e.astype(jnp.float32) * LN2 + (2.0 * s + s * p)


def _body(scores_h, labels_h, ids_h, out_h,
          ids_buf, sc_buf, lb_buf,
          tab, btab, dv, ctab,
          part, allpart, outv,
          sh_tabs, sh_comb, sh_part):
    cid = lax.axis_index("c")
    ws = lax.axis_index("s")
    active = cid == 0
    base = ws * CHUNK
    lane = lax.iota(jnp.int32, 16)
    zeros = jnp.zeros((16,), jnp.float32)

    def phase_a():
        pltpu.sync_copy(ids_h.at[pl.ds(base, CHUNK)], ids_buf)
        pltpu.sync_copy(scores_h.at[pl.ds(base, CHUNK)], sc_buf)
        pltpu.sync_copy(labels_h.at[pl.ds(base, CHUNK)], lb_buf)

        def zero_tab(z, _):
            tab[pl.ds(z * 16, 16)] = zeros
            return 0
        lax.fori_loop(0, NBLK * ABLK // 16, zero_tab, 0)

        def step(i, carry):
            prev, ste, sse, cnt = carry
            idxv = lane * STRIPE + i
            idv = plsc.load_gather(ids_buf, [idxv])
            tev = jnp.exp(plsc.load_gather(lb_buf, [idxv]))
            sev = jnp.exp(plsc.load_gather(sc_buf, [idxv]))
            changed = idv != prev
            fl = changed & (prev >= 0.0)
            pidx = jnp.maximum(prev, 0.0).astype(jnp.int32)
            poff = pidx + lax.shift_left(
                lax.shift_right_logical(pidx, 6), 7)
            plsc.addupdate_scatter(tab, [poff], cnt, mask=fl)
            plsc.addupdate_scatter(tab, [poff + QPW], ste, mask=fl)
            plsc.addupdate_scatter(tab, [poff + 2 * QPW], sse, mask=fl)
            ste = jnp.where(changed, tev, ste + tev)
            sse = jnp.where(changed, sev, sse + sev)
            cnt = jnp.where(changed, 1.0, cnt + 1.0)
            return idv, ste, sse, cnt

        prev, ste, sse, cnt = lax.fori_loop(
            0, STRIPE, step,
            (jnp.full((16,), -1.0, jnp.float32), zeros, zeros, zeros))
        pidx = prev.astype(jnp.int32)
        poff = pidx + lax.shift_left(lax.shift_right_logical(pidx, 6), 7)
        for t in range(16):
            m = lane == t
            plsc.addupdate_scatter(tab, [poff], cnt, mask=m)
            plsc.addupdate_scatter(tab, [poff + QPW], ste, mask=m)
            plsc.addupdate_scatter(tab, [poff + 2 * QPW], sse, mask=m)

        pltpu.sync_copy(tab, sh_tabs.at[pl.ds(ws * NBLK * ABLK, NBLK * ABLK)])

    pl.when(active)(phase_a)
    plsc.subcore_barrier()

    def phase_b():
        for t in range(NSUB):
            pltpu.sync_copy(
                sh_tabs.at[pl.ds(t * NBLK * ABLK + ws * ABLK, ABLK)],
                btab.at[pl.ds(t * ABLK, ABLK)])
        nv = zeros
        for j in range(QPW // 16):
            o = j * 16
            cnt = zeros
            tden = zeros
            sden = zeros
            for t in range(NSUB):
                cnt = cnt + btab[pl.ds(t * ABLK + o, 16)]
                tden = tden + btab[pl.ds(t * ABLK + QPW + o, 16)]
                sden = sden + btab[pl.ds(t * ABLK + 2 * QPW + o, 16)]
            validf = jnp.where(cnt >= 2.0, 1.0, 0.0)
            nv = nv + validf
            dv[pl.ds(o, 16)] = (1.0 - SMOOTH) / tden
            dv[pl.ds(QPW + o, 16)] = SMOOTH / jnp.maximum(cnt, 1.0)
            dv[pl.ds(2 * QPW + o, 16)] = _vlog(sden)
            dv[pl.ds(3 * QPW + o, 16)] = validf
        part[pl.ds(16, 16)] = nv
        pltpu.sync_copy(dv, sh_comb.at[pl.ds(ws * CBLK, CBLK)])

    pl.when(active)(phase_b)
    plsc.subcore_barrier()

    def phase_c():
        b0 = ids_buf[pl.ds(0, 16)][0].astype(jnp.int32) // QPW
        b1 = ids_buf[pl.ds(CHUNK - 16, 16)][15].astype(jnp.int32) // QPW
        for b in range(NBLK):
            def get_blk(b=b):
                pltpu.sync_copy(sh_comb.at[pl.ds(b * CBLK, CBLK)],
                                ctab.at[pl.ds((b - b0) * CBLK, CBLK)])
            pl.when((b0 <= b) & (b <= b1))(get_blk)

        shift = b0 * CBLK

        def vec(v, acc):
            o = v * 16
            idv = ids_buf[pl.ds(o, 16)].astype(jnp.int32)
            tev = jnp.exp(lb_buf[pl.ds(o, 16)])
            scv = sc_buf[pl.ds(o, 16)]
            coff = idv + lax.shift_right_logical(idv, 6) * 192 - shift
            r_g = plsc.load_gather(ctab, [coff])
            s_g = plsc.load_gather(ctab, [coff + QPW])
            l_g = plsc.load_gather(ctab, [coff + 2 * QPW])
            v_g = plsc.load_gather(ctab, [coff + 3 * QPW])
            target = r_g * tev + s_g
            kl = target * (_vlog(target) - scv + l_g)
            return acc + kl * v_g
        acc = lax.fori_loop(0, VECS, vec, zeros)
        part[pl.ds(0, 16)] = acc
        pltpu.sync_copy(part, sh_part.at[pl.ds(ws * 32, 32)])

    pl.when(active)(phase_c)
    plsc.subcore_barrier()

    def final():
        pltpu.sync_copy(sh_part, allpart)
        tot = zeros
        nvv = zeros
        for t in range(NSUB):
            tot = tot + allpart[pl.ds(t * 32, 16)]
            nvv = nvv + allpart[pl.ds(t * 32 + 16, 16)]
        part[pl.ds(0, 16)] = tot
        part[pl.ds(16, 16)] = nvv
        tsum = zeros
        nsum = zeros
        for t in range(16):
            ix = jnp.full((16,), t, jnp.int32)
            tsum = tsum + plsc.load_gather(part, [ix])
            nsum = nsum + plsc.load_gather(part, [ix + 16])
        outv[...] = tsum / jnp.maximum(nsum, 1.0)
        pltpu.sync_copy(outv, out_h)

    pl.when(active & (ws == 0))(final)


_mesh = plsc.VectorSubcoreMesh(core_axis_name="c", subcore_axis_name="s")

_sc_call = functools.partial(
    pl.kernel,
    out_type=jax.ShapeDtypeStruct((16,), jnp.float32),
    mesh=_mesh,
    compiler_params=pltpu.CompilerParams(needs_layout_passes=False),
    scratch_types=[
        pltpu.VMEM((CHUNK,), jnp.float32),
        pltpu.VMEM((CHUNK,), jnp.float32),
        pltpu.VMEM((CHUNK,), jnp.float32),
        pltpu.VMEM((NBLK * ABLK,), jnp.float32),
        pltpu.VMEM((NSUB * ABLK,), jnp.float32),
        pltpu.VMEM((CBLK,), jnp.float32),
        pltpu.VMEM((NBLK * CBLK,), jnp.float32),
        pltpu.VMEM((32,), jnp.float32),
        pltpu.VMEM((NSUB * 32,), jnp.float32),
        pltpu.VMEM((16,), jnp.float32),
        pltpu.VMEM_SHARED((NSUB * NBLK * ABLK,), jnp.float32),
        pltpu.VMEM_SHARED((NSUB * CBLK,), jnp.float32),
        pltpu.VMEM_SHARED((NSUB * 32,), jnp.float32),
    ],
)(_body)


def kernel(scores, labels, query_ids):
    out = _sc_call(scores, labels, query_ids.astype(jnp.float32))
    return out[0]


# speedup vs baseline: 1.0692x; 1.0692x over previous
import functools

import jax
import jax.numpy as jnp
from jax import lax
from jax.experimental import pallas as pl
from jax.experimental.pallas import tpu as pltpu
from jax.experimental.pallas import tpu_sc as plsc

N = 16384
Q = 1024
NSUB = 16
CHUNK = N // NSUB
VECS = CHUNK // 16
STRIPE = CHUNK // 16
QPW = Q // NSUB
NBLK = Q // QPW
ABLK = 3 * QPW
CBLK = 4 * QPW
SMOOTH = 0.1
LN2 = 0.6931471805599453


def _vlog(x):
    xi = lax.bitcast_convert_type(x, jnp.int32)
    e = lax.shift_right_arithmetic(xi, 23) - 127
    m = lax.bitcast_convert_type(
        (xi & 0x007FFFFF) | 0x3F800000, jnp.float32)
    big = m >= 1.4142135623730951
    m = jnp.where(big, m * 0.5, m)
    e = jnp.where(big, e + 1, e)
    f = m - 1.0
    s = f / (2.0 + f)
    w = s * s
    p = w * (0.6666666666666735 + w * (0.3999999999940942
         + w * (0.2857142874366239 + w * 0.22222198432149784)))
    return e.astype(jnp.float32) * LN2 + (2.0 * s + s * p)


def _body(scores_h, labels_h, ids_h, out_h,
          ids_buf, sc_buf, lb_buf,
          tab, btab, dv, ctab,
          part, allpart, outv, sem,
          sh_tabs, sh_comb, sh_part):
    cid = lax.axis_index("c")
    ws = lax.axis_index("s")
    active = cid == 0
    base = ws * CHUNK
    lane = lax.iota(jnp.int32, 16)
    zeros = jnp.zeros((16,), jnp.float32)

    def phase_a():
        c1 = pltpu.async_copy(ids_h.at[pl.ds(base, CHUNK)], ids_buf, sem)
        c2 = pltpu.async_copy(scores_h.at[pl.ds(base, CHUNK)], sc_buf, sem)
        c3 = pltpu.async_copy(labels_h.at[pl.ds(base, CHUNK)], lb_buf, sem)

        def zero_tab(z, _):
            tab[pl.ds(z * 16, 16)] = zeros
            return 0
        lax.fori_loop(0, NBLK * ABLK // 16, zero_tab, 0)
        c1.wait()
        c2.wait()
        c3.wait()

        def step(i, carry):
            prev, ste, sse, cnt = carry
            idxv = lane * STRIPE + i
            idv = plsc.load_gather(ids_buf, [idxv])
            tev = jnp.exp(plsc.load_gather(lb_buf, [idxv]))
            sev = jnp.exp(plsc.load_gather(sc_buf, [idxv]))
            changed = idv != prev
            fl = changed & (prev >= 0.0)
            pidx = jnp.maximum(prev, 0.0).astype(jnp.int32)
            poff = pidx + lax.shift_left(
                lax.shift_right_logical(pidx, 6), 7)
            plsc.addupdate_scatter(tab, [poff], cnt, mask=fl)
            plsc.addupdate_scatter(tab, [poff + QPW], ste, mask=fl)
            plsc.addupdate_scatter(tab, [poff + 2 * QPW], sse, mask=fl)
            ste = jnp.where(changed, tev, ste + tev)
            sse = jnp.where(changed, sev, sse + sev)
            cnt = jnp.where(changed, 1.0, cnt + 1.0)
            return idv, ste, sse, cnt

        prev, ste, sse, cnt = lax.fori_loop(
            0, STRIPE, step,
            (jnp.full((16,), -1.0, jnp.float32), zeros, zeros, zeros))
        pidx = prev.astype(jnp.int32)
        poff = pidx + lax.shift_left(lax.shift_right_logical(pidx, 6), 7)
        for t in range(16):
            m = lane == t
            plsc.addupdate_scatter(tab, [poff], cnt, mask=m)
            plsc.addupdate_scatter(tab, [poff + QPW], ste, mask=m)
            plsc.addupdate_scatter(tab, [poff + 2 * QPW], sse, mask=m)

        pltpu.sync_copy(tab, sh_tabs.at[pl.ds(ws * NBLK * ABLK, NBLK * ABLK)])

    pl.when(active)(phase_a)
    plsc.subcore_barrier()

    def phase_b():
        for t in range(NSUB):
            pltpu.sync_copy(
                sh_tabs.at[pl.ds(t * NBLK * ABLK + ws * ABLK, ABLK)],
                btab.at[pl.ds(t * ABLK, ABLK)])
        nv = zeros
        for j in range(QPW // 16):
            o = j * 16
            cnt = zeros
            tden = zeros
            sden = zeros
            for t in range(NSUB):
                cnt = cnt + btab[pl.ds(t * ABLK + o, 16)]
                tden = tden + btab[pl.ds(t * ABLK + QPW + o, 16)]
                sden = sden + btab[pl.ds(t * ABLK + 2 * QPW + o, 16)]
            validf = jnp.where(cnt >= 2.0, 1.0, 0.0)
            nv = nv + validf
            dv[pl.ds(o, 16)] = (1.0 - SMOOTH) / tden
            dv[pl.ds(QPW + o, 16)] = SMOOTH / jnp.maximum(cnt, 1.0)
            dv[pl.ds(2 * QPW + o, 16)] = _vlog(sden)
            dv[pl.ds(3 * QPW + o, 16)] = validf
        part[pl.ds(16, 16)] = nv
        pltpu.sync_copy(dv, sh_comb.at[pl.ds(ws * CBLK, CBLK)])

    pl.when(active)(phase_b)
    plsc.subcore_barrier()

    def phase_c():
        b0 = ids_buf[pl.ds(0, 16)][0].astype(jnp.int32) // QPW
        b1 = ids_buf[pl.ds(CHUNK - 16, 16)][15].astype(jnp.int32) // QPW
        for b in range(NBLK):
            def get_blk(b=b):
                pltpu.sync_copy(sh_comb.at[pl.ds(b * CBLK, CBLK)],
                                ctab.at[pl.ds((b - b0) * CBLK, CBLK)])
            pl.when((b0 <= b) & (b <= b1))(get_blk)

        shift = b0 * CBLK

        def vec(v, acc):
            o = v * 16
            idv = ids_buf[pl.ds(o, 16)].astype(jnp.int32)
            tev = jnp.exp(lb_buf[pl.ds(o, 16)])
            scv = sc_buf[pl.ds(o, 16)]
            coff = idv + lax.shift_right_logical(idv, 6) * 192 - shift
            r_g = plsc.load_gather(ctab, [coff])
            s_g = plsc.load_gather(ctab, [coff + QPW])
            l_g = plsc.load_gather(ctab, [coff + 2 * QPW])
            v_g = plsc.load_gather(ctab, [coff + 3 * QPW])
            target = r_g * tev + s_g
            kl = target * (_vlog(target) - scv + l_g)
            return acc + kl * v_g
        acc = lax.fori_loop(0, VECS, vec, zeros)
        part[pl.ds(0, 16)] = acc
        pltpu.sync_copy(part, sh_part.at[pl.ds(ws * 32, 32)])

    pl.when(active)(phase_c)
    plsc.subcore_barrier()

    def final():
        pltpu.sync_copy(sh_part, allpart)
        tot = zeros
        nvv = zeros
        for t in range(NSUB):
            tot = tot + allpart[pl.ds(t * 32, 16)]
            nvv = nvv + allpart[pl.ds(t * 32 + 16, 16)]
        part[pl.ds(0, 16)] = tot
        part[pl.ds(16, 16)] = nvv
        tsum = zeros
        nsum = zeros
        for t in range(16):
            ix = jnp.full((16,), t, jnp.int32)
            tsum = tsum + plsc.load_gather(part, [ix])
            nsum = nsum + plsc.load_gather(part, [ix + 16])
        outv[...] = tsum / jnp.maximum(nsum, 1.0)
        pltpu.sync_copy(outv, out_h)

    pl.when(active & (ws == 0))(final)


_mesh = plsc.VectorSubcoreMesh(core_axis_name="c", subcore_axis_name="s")

_sc_call = functools.partial(
    pl.kernel,
    out_type=jax.ShapeDtypeStruct((16,), jnp.float32),
    mesh=_mesh,
    compiler_params=pltpu.CompilerParams(needs_layout_passes=False),
    scratch_types=[
        pltpu.VMEM((CHUNK,), jnp.float32),
        pltpu.VMEM((CHUNK,), jnp.float32),
        pltpu.VMEM((CHUNK,), jnp.float32),
        pltpu.VMEM((NBLK * ABLK,), jnp.float32),
        pltpu.VMEM((NSUB * ABLK,), jnp.float32),
        pltpu.VMEM((CBLK,), jnp.float32),
        pltpu.VMEM((NBLK * CBLK,), jnp.float32),
        pltpu.VMEM((32,), jnp.float32),
        pltpu.VMEM((NSUB * 32,), jnp.float32),
        pltpu.VMEM((16,), jnp.float32),
        pltpu.SemaphoreType.DMA,
        pltpu.VMEM_SHARED((NSUB * NBLK * ABLK,), jnp.float32),
        pltpu.VMEM_SHARED((NSUB * CBLK,), jnp.float32),
        pltpu.VMEM_SHARED((NSUB * 32,), jnp.float32),
    ],
)(_body)


def kernel(scores, labels, query_ids):
    out = _sc_call(scores, labels, query_ids.astype(jnp.float32))
    return out[0]
